# Initial kernel scaffold; baseline (speedup 1.0000x reference)
import jax
import jax.numpy as jnp
from jax.experimental import pallas as pl


def kernel(pred_boxes, pred_classes, gt_boxes, gt_classes, tgt_pair_idxs, tgt_rel_labs, rel_prop_pair_idx):
    raise NotImplementedError("write your pallas kernel here")


# speedup vs baseline: 42.5368x; 42.5368x over previous
import jax
import jax.numpy as jnp
from jax.experimental import pallas as pl


_MCHUNK = 512


def _kern(pbT_ref, gb_ref, tp_ref, rpT_ref, corr_ref, mq_ref, br_ref):
    f32 = jnp.float32
    pbT = pbT_ref[:]
    gb = gb_ref[:]
    N = pbT.shape[1]
    G = gb.shape[0]
    R = tp_ref.shape[0]
    M = rpT_ref.shape[1]

    gx1, gy1, gx2, gy2 = gb[:, 0:1], gb[:, 1:2], gb[:, 2:3], gb[:, 3:4]
    px1, py1, px2, py2 = pbT[0:1, :], pbT[1:2, :], pbT[2:3, :], pbT[3:4, :]
    a1 = (gx2 - gx1) * (gy2 - gy1)
    a2 = (px2 - px1) * (py2 - py1)
    ltx = jnp.maximum(gx1, px1)
    lty = jnp.maximum(gy1, py1)
    rbx = jnp.minimum(gx2, px2)
    rby = jnp.minimum(gy2, py2)
    w = jnp.clip(rbx - ltx, 0.0)
    h = jnp.clip(rby - lty, 0.0)
    inter = w * h
    union = a1 + a2 - inter
    ious = jnp.where(union > 0, inter / jnp.maximum(union, 1e-9), 0.0)

    match = (ious > 0.5).astype(f32)
    pgi = jnp.max(ious, axis=0, keepdims=True)

    head = tp_ref[:, 0:1]
    tail = tp_ref[:, 1:2]
    iota_g = jax.lax.broadcasted_iota(jnp.int32, (R, G), 1)
    oh_head = (head == iota_g).astype(f32)
    oh_tail = (tail == iota_g).astype(f32)
    Hm = jnp.dot(oh_head, match, preferred_element_type=f32)
    Tm = jnp.dot(oh_tail, match, preferred_element_type=f32)

    dn = (((0,), (0,)), ((), ()))
    A = jax.lax.dot_general(Hm, Tm, dn, preferred_element_type=f32)
    A2 = jax.lax.dot_general(Tm, Hm, dn, preferred_element_type=f32)
    br_ref[:] = (A + A2 > 0.0).astype(jnp.int32)

    iota_r = jax.lax.broadcasted_iota(jnp.int32, (R, _MCHUNK), 0)
    iota_n = jax.lax.broadcasted_iota(jnp.int32, (N, _MCHUNK), 0)
    for c in range(M // _MCHUNK):
        hc = rpT_ref[0:1, c * _MCHUNK:(c + 1) * _MCHUNK]
        tc = rpT_ref[1:2, c * _MCHUNK:(c + 1) * _MCHUNK]
        oh_h = (iota_n == hc).astype(f32)
        oh_t = (iota_n == tc).astype(f32)
        Hh = jnp.dot(Hm, oh_h, preferred_element_type=f32)
        Tt = jnp.dot(Tm, oh_t, preferred_element_type=f32)
        cond = (Hh > 0.5) & (Tt > 0.5)
        corr = jnp.max(jnp.where(cond, iota_r, -1), axis=0, keepdims=True)
        corr = jnp.where(hc == tc, -1, corr)
        corr_ref[c:c + 1, :] = corr
        ph = jnp.dot(pgi, oh_h, preferred_element_type=f32)
        pt = jnp.dot(pgi, oh_t, preferred_element_type=f32)
        mq_ref[c:c + 1, :] = ph * pt


def kernel(pred_boxes, pred_classes, gt_boxes, gt_classes, tgt_pair_idxs,
           tgt_rel_labs, rel_prop_pair_idx):
    N = pred_boxes.shape[0]
    M = rel_prop_pair_idx.shape[0]
    pbT = pred_boxes.T
    rpT = rel_prop_pair_idx.T.astype(jnp.int32)
    nchunks = M // _MCHUNK
    corr, mq, br = pl.pallas_call(
        _kern,
        out_shape=(
            jax.ShapeDtypeStruct((nchunks, _MCHUNK), jnp.int32),
            jax.ShapeDtypeStruct((nchunks, _MCHUNK), jnp.float32),
            jax.ShapeDtypeStruct((N, N), jnp.int32),
        ),
    )(pbT, gt_boxes, tgt_pair_idxs.astype(jnp.int32), rpT)
    return corr.reshape(M), mq.reshape(M), br


# speedup vs baseline: 53.9451x; 1.2682x over previous
import jax
import jax.numpy as jnp
from jax.experimental import pallas as pl


_MCHUNK = 512
_NWORDS = 16


def _kern(pbT_ref, gb_ref, tp_ref, rpT_ref, corr_ref, mq_ref, br_ref):
    f32 = jnp.float32
    i32 = jnp.int32
    pbT = pbT_ref[:]
    gb = gb_ref[:]
    N = pbT.shape[1]
    G = gb.shape[0]
    R = tp_ref.shape[0]
    M = rpT_ref.shape[1]

    gx1, gy1, gx2, gy2 = gb[:, 0:1], gb[:, 1:2], gb[:, 2:3], gb[:, 3:4]
    px1, py1, px2, py2 = pbT[0:1, :], pbT[1:2, :], pbT[2:3, :], pbT[3:4, :]
    a1 = (gx2 - gx1) * (gy2 - gy1)
    a2 = (px2 - px1) * (py2 - py1)
    w = jnp.clip(jnp.minimum(gx2, px2) - jnp.maximum(gx1, px1), 0.0)
    h = jnp.clip(jnp.minimum(gy2, py2) - jnp.maximum(gy1, py1), 0.0)
    inter = w * h
    union = a1 + a2 - inter
    ious = jnp.where(union > 0, inter / jnp.maximum(union, 1e-9), 0.0)

    match = (ious > 0.5).astype(f32)
    pgi = jnp.max(ious, axis=0, keepdims=True)

    head = tp_ref[:, 0:1]
    tail = tp_ref[:, 1:2]
    iota_g = jax.lax.broadcasted_iota(i32, (R, G), 1)
    oh_head = (head == iota_g).astype(f32)
    oh_tail = (tail == iota_g).astype(f32)

    dn0 = (((0,), (0,)), ((), ()))
    P1 = jax.lax.dot_general(oh_head, oh_tail, dn0, preferred_element_type=f32)
    P2 = jax.lax.dot_general(oh_tail, oh_head, dn0, preferred_element_type=f32)
    Q = P1 + P2
    Qm = jnp.dot(Q, match, preferred_element_type=f32)
    B = jax.lax.dot_general(match.astype(jnp.bfloat16), Qm.astype(jnp.bfloat16),
                            dn0, preferred_element_type=f32)
    br_ref[:] = (B > 0.0).astype(i32)

    ci = jax.lax.broadcasted_iota(i32, (_NWORDS, R), 0)
    ri = jax.lax.broadcasted_iota(i32, (_NWORDS, R), 1)
    wpack = jnp.where(ri // 16 == ci, (1 << (ri % 16)), 0).astype(f32)
    wh = jnp.dot(wpack, oh_head, preferred_element_type=f32)
    wt = jnp.dot(wpack, oh_tail, preferred_element_type=f32)
    pwh = jnp.dot(wh, match, preferred_element_type=f32)
    pwt = jnp.dot(wt, match, preferred_element_type=f32)
    S = jnp.concatenate([pwh, pwt, pgi], axis=0)

    iota_w = jax.lax.broadcasted_iota(i32, (_NWORDS, _MCHUNK), 0)
    iota_n2 = jax.lax.broadcasted_iota(i32, (N, 2 * _MCHUNK), 0)
    for c in range(M // _MCHUNK):
        hc = rpT_ref[0:1, c * _MCHUNK:(c + 1) * _MCHUNK]
        tc = rpT_ref[1:2, c * _MCHUNK:(c + 1) * _MCHUNK]
        ht = jnp.concatenate([hc, tc], axis=1)
        oh = (iota_n2 == ht).astype(f32)
        gat = jnp.dot(S, oh, preferred_element_type=f32)
        and_w = (gat[0:_NWORDS, 0:_MCHUNK].astype(i32)
                 & gat[_NWORDS:2 * _NWORDS, _MCHUNK:2 * _MCHUNK].astype(i32))
        e = (jax.lax.bitcast_convert_type(and_w.astype(f32), i32) >> 23) - 127
        val = jnp.where(and_w > 0, e + 16 * iota_w, -1)
        corr = jnp.max(val, axis=0, keepdims=True)
        corr = jnp.where(hc == tc, -1, corr)
        corr_ref[c:c + 1, :] = corr
        ph = gat[2 * _NWORDS:2 * _NWORDS + 1, 0:_MCHUNK]
        pt = gat[2 * _NWORDS:2 * _NWORDS + 1, _MCHUNK:2 * _MCHUNK]
        mq_ref[c:c + 1, :] = ph * pt


def kernel(pred_boxes, pred_classes, gt_boxes, gt_classes, tgt_pair_idxs,
           tgt_rel_labs, rel_prop_pair_idx):
    N = pred_boxes.shape[0]
    M = rel_prop_pair_idx.shape[0]
    pbT = pred_boxes.T
    rpT = rel_prop_pair_idx.T.astype(jnp.int32)
    nchunks = M // _MCHUNK
    corr, mq, br = pl.pallas_call(
        _kern,
        out_shape=(
            jax.ShapeDtypeStruct((nchunks, _MCHUNK), jnp.int32),
            jax.ShapeDtypeStruct((nchunks, _MCHUNK), jnp.float32),
            jax.ShapeDtypeStruct((N, N), jnp.int32),
        ),
    )(pbT, gt_boxes, tgt_pair_idxs.astype(jnp.int32), rpT)
    return corr.reshape(M), mq.reshape(M), br
